# baseline (speedup 1.0000x reference)
import dataclasses
import functools

import jax
import jax.numpy as jnp
from jax import lax
from jax.experimental import pallas as pl
from jax.experimental.pallas import tpu as pltpu
from jax.experimental.pallas import tpu_sc as plsc

T = 2048
D = 768
H = 1024
E = 8
CAP = 640
NSLOT = E * CAP
DISP_PAD = NSLOT + 16
NW = 32
_PREC = lax.Precision.DEFAULT


def _router_body(lg_ref, dest_ref, destc_ref, scale_ref, aux_ref,
                 util_ref):
    logits = lg_ref[...]
    m = jnp.max(logits, axis=1, keepdims=True)
    p = jnp.exp(logits - m)
    probs = p / jnp.sum(p, axis=1, keepdims=True)

    eio = lax.broadcasted_iota(jnp.int32, (T, E), 1)
    v1 = jnp.max(probs, axis=1, keepdims=True)
    a1 = jnp.min(jnp.where(probs == v1, eio, E), axis=1, keepdims=True)
    m1 = eio == a1
    pm = jnp.where(m1, -1.0, probs)
    v2 = jnp.max(pm, axis=1, keepdims=True)
    a2 = jnp.min(jnp.where(pm == v2, eio, E), axis=1, keepdims=True)
    m2 = eio == a2
    oh = jnp.logical_or(m1, m2)

    counts = jnp.sum(oh.astype(jnp.float32), axis=0, keepdims=True)
    util = jnp.sum((counts > 0.0).astype(jnp.float32)) * (100.0 / E)
    meanprob = jnp.sum(probs, axis=0, keepdims=True) * (1.0 / T)
    aux = E * jnp.sum(meanprob * (counts * (1.0 / T)))
    util_ref[...] = jnp.reshape(util, (1, 1))
    aux_ref[...] = jnp.reshape(aux, (1, 1))

    pos = oh.astype(jnp.int32)
    sh = 1
    while sh < T:
        shifted = jnp.concatenate(
            [jnp.zeros((sh, E), jnp.int32), pos[: T - sh]], axis=0)
        pos = pos + shifted
        sh *= 2
    kept = jnp.logical_and(oh, pos <= CAP)
    slot = eio * CAP + (pos - 1)

    wsum = v1 + v2
    s1 = v1 / wsum
    s2 = v2 / wsum

    def pick(mk):
        sel = jnp.logical_and(mk, kept)
        d = jnp.sum(jnp.where(sel, slot, 0), axis=1, keepdims=True)
        k = jnp.sum(sel.astype(jnp.int32), axis=1, keepdims=True) > 0
        return d, k

    d1, k1 = pick(m1)
    d2, k2 = pick(m2)
    dest_ref[...] = jnp.concatenate(
        [jnp.where(k1, d1, NSLOT), jnp.where(k2, d2, NSLOT)], axis=1)
    destc_ref[...] = jnp.concatenate(
        [jnp.where(k1, d1, 0), jnp.where(k2, d2, 0)], axis=1)
    scale_ref[...] = jnp.concatenate(
        [jnp.where(k1, s1, 0.0), jnp.where(k2, s2, 0.0)], axis=1)


def _router(logits):
    return pl.pallas_call(
        _router_body,
        out_shape=[
            jax.ShapeDtypeStruct((T, 2), jnp.int32),
            jax.ShapeDtypeStruct((T, 2), jnp.int32),
            jax.ShapeDtypeStruct((T, 2), jnp.float32),
            jax.ShapeDtypeStruct((1, 1), jnp.float32),
            jax.ShapeDtypeStruct((1, 1), jnp.float32),
        ],
    )(logits)


def _build_disp(dest_flat):
    mesh = plsc.VectorSubcoreMesh(core_axis_name="c", subcore_axis_name="s")
    cp = pltpu.CompilerParams()
    if "needs_layout_passes" in pltpu.CompilerParams.__dataclass_fields__:
        cp = dataclasses.replace(cp, needs_layout_passes=False)

    @functools.partial(
        pl.kernel,
        out_type=jax.ShapeDtypeStruct((NSLOT,), jnp.int32),
        mesh=mesh,
        compiler_params=cp,
        scratch_types=[
            pltpu.VMEM((2 * T,), jnp.int32),
            pltpu.VMEM((DISP_PAD,), jnp.int32),
        ],
    )
    def k(dest_hbm, disp_hbm, dest_v, disp_v):
        wid = lax.axis_index("s") * 2 + lax.axis_index("c")

        @pl.when(wid == 0)
        def _():
            pltpu.sync_copy(dest_hbm, dest_v)

            @pl.loop(0, DISP_PAD // 16)
            def _(i):
                disp_v[pl.ds(i * 16, 16)] = (
                    (i * 16 + lax.iota(jnp.int32, 16)) & (T - 1))

            @pl.loop(0, (2 * T) // 16)
            def _(i):
                idx16 = dest_v[pl.ds(i * 16, 16)]
                tok16 = lax.shift_right_logical(
                    i * 16 + lax.iota(jnp.int32, 16), 1)
                plsc.store_scatter(disp_v, [idx16], tok16)

            pltpu.sync_copy(disp_v.at[pl.ds(0, NSLOT)], disp_hbm)

    return k(dest_flat)


def _sc_gather(table, idx, nchunks):
    nrows, d = idx.shape[0], table.shape[1]
    per_w = nrows // NW
    chunk = per_w // nchunks
    assert chunk * nchunks == per_w and chunk <= 128 and chunk % 8 == 0
    mesh = plsc.VectorSubcoreMesh(core_axis_name="c", subcore_axis_name="s")

    @functools.partial(
        pl.kernel,
        out_type=jax.ShapeDtypeStruct((nrows, d), jnp.float32),
        mesh=mesh,
        scratch_types=(
            [pltpu.VMEM((nchunks, chunk), jnp.int32)]
            + [pltpu.VMEM((chunk, d), jnp.float32)] * nchunks
            + [pltpu.SemaphoreType.DMA] * (2 * nchunks)
        ),
    )
    def k(table_hbm, idx_hbm, out_hbm, idx_v, *bufs_and_sems):
        rows = bufs_and_sems[:nchunks]
        sg = bufs_and_sems[nchunks:2 * nchunks]
        so = bufs_and_sems[2 * nchunks:]
        wid = lax.axis_index("s") * 2 + lax.axis_index("c")
        base = wid * per_w
        for j in range(nchunks):
            pltpu.sync_copy(idx_hbm.at[pl.ds(base + j * chunk, chunk)],
                            idx_v.at[j])
        gs = [pltpu.async_copy(table_hbm.at[idx_v.at[j]], rows[j], sg[j])
              for j in range(nchunks)]
        os = []
        for j in range(nchunks):
            gs[j].wait()
            os.append(pltpu.async_copy(
                rows[j], out_hbm.at[pl.ds(base + j * chunk, chunk)], so[j]))
        for o in os:
            o.wait()

    return k(table, idx)


_HD = D // 2


def _pack_halves(a, b):
    ua = lax.bitcast_convert_type(a, jnp.uint32)
    ub = lax.bitcast_convert_type(b, jnp.uint32)
    pa = lax.shift_right_logical(ua + jnp.uint32(0x8000), jnp.uint32(16))
    pb = (ub + jnp.uint32(0x8000)) & jnp.uint32(0xFFFF0000)
    return lax.bitcast_convert_type(pa | pb, jnp.float32)


def _unpack_halves(p):
    up = lax.bitcast_convert_type(p, jnp.uint32)
    a = lax.bitcast_convert_type(lax.shift_left(up, jnp.uint32(16)),
                                 jnp.float32)
    b = lax.bitcast_convert_type(up & jnp.uint32(0xFFFF0000), jnp.float32)
    return jnp.concatenate([a, b], axis=1)


def _pack_x_body(x_ref, o_ref):
    xv = x_ref[...]
    o_ref[...] = _pack_halves(xv[:, :_HD], xv[:, _HD:])


def _pack_x(x):
    return pl.pallas_call(
        _pack_x_body,
        out_shape=jax.ShapeDtypeStruct((T, _HD), jnp.float32),
    )(x)


def _swiglu_block(xb, w1, w2, w3):
    h1 = lax.dot_general(xb, w1, (((1,), (1,)), ((), ())),
                         preferred_element_type=jnp.float32, precision=_PREC)
    h3 = lax.dot_general(xb, w3, (((1,), (1,)), ((), ())),
                         preferred_element_type=jnp.float32, precision=_PREC)
    g = h1 * (1.0 / (1.0 + jnp.exp(-h1))) * h3
    return lax.dot_general(g, w2, (((1,), (1,)), ((), ())),
                           preferred_element_type=jnp.float32,
                           precision=_PREC)


def _expert_ffn_body(x_ref, w1_ref, w2_ref, w3_ref, o_ref, acc):
    xb = _unpack_halves(x_ref[0])
    yp = _swiglu_block(xb, w1_ref[0], w2_ref[0], w3_ref[0])

    @pl.when(pl.program_id(1) == 0)
    def _():
        acc[...] = yp

    @pl.when(pl.program_id(1) == 1)
    def _():
        y = acc[...] + yp
        o_ref[0] = _pack_halves(y[:, :_HD], y[:, _HD:])


def _expert_ffn(xbufh3, w1s, w2s, w3s):
    return pl.pallas_call(
        _expert_ffn_body,
        grid=(E, 2),
        in_specs=[
            pl.BlockSpec((1, CAP, _HD), lambda e, h: (e, 0, 0)),
            pl.BlockSpec((1, H // 2, D), lambda e, h: (e, h, 0)),
            pl.BlockSpec((1, D, H // 2), lambda e, h: (e, 0, h)),
            pl.BlockSpec((1, H // 2, D), lambda e, h: (e, h, 0)),
        ],
        out_specs=pl.BlockSpec((1, CAP, _HD), lambda e, h: (e, 0, 0)),
        out_shape=jax.ShapeDtypeStruct((E, CAP, _HD), jnp.float32),
        scratch_shapes=[pltpu.VMEM((CAP, D), jnp.float32)],
    )(xbufh3, w1s, w2s, w3s)


def _shared_ffn_body(x_ref, w1_ref, w2_ref, w3_ref, o_ref):
    o_ref[...] = _swiglu_block(x_ref[...], w1_ref[...], w2_ref[...],
                               w3_ref[...])


def _shared_ffn(x, sw1, sw2, sw3):
    nb = 8
    return pl.pallas_call(
        _shared_ffn_body,
        grid=(nb,),
        in_specs=[
            pl.BlockSpec((T // nb, D), lambda i: (i, 0)),
            pl.BlockSpec((H, D), lambda i: (0, 0)),
            pl.BlockSpec((D, H), lambda i: (0, 0)),
            pl.BlockSpec((H, D), lambda i: (0, 0)),
        ],
        out_specs=pl.BlockSpec((T // nb, D), lambda i: (i, 0)),
        out_shape=jax.ShapeDtypeStruct((T, D), jnp.float32),
    )(x, sw1, sw2, sw3)


def _combine_body(g_ref, sh_ref, s0_ref, s1_ref, o_ref):
    g = g_ref[...]
    g0 = _unpack_halves(g[:, :_HD])
    g1 = _unpack_halves(g[:, _HD:])
    o_ref[...] = sh_ref[...] + s0_ref[...] * g0 + s1_ref[...] * g1


def _combine(g2, shared, s0, s1):
    nb = 4
    return pl.pallas_call(
        _combine_body,
        grid=(nb,),
        in_specs=[
            pl.BlockSpec((T // nb, D), lambda i: (i, 0)),
            pl.BlockSpec((T // nb, D), lambda i: (i, 0)),
            pl.BlockSpec((T // nb, 1), lambda i: (i, 0)),
            pl.BlockSpec((T // nb, 1), lambda i: (i, 0)),
        ],
        out_specs=pl.BlockSpec((T // nb, D), lambda i: (i, 0)),
        out_shape=jax.ShapeDtypeStruct((T, D), jnp.float32),
    )(g2, shared, s0, s1)


def kernel(x_in, router_w, w1s, w2s, w3s, sw1, sw2, sw3):
    x = x_in.reshape(T, D)
    logits = x @ router_w.T
    dest, destc, scale, aux, util = _router(logits)
    disp = _build_disp(dest.reshape(-1))
    xh = _pack_x(x)
    xbufh = _sc_gather(xh, disp, nchunks=4)
    shared = _shared_ffn(x, sw1, sw2, sw3)
    yh = _expert_ffn(xbufh.reshape(E, CAP, _HD), w1s, w2s, w3s)
    gh = _sc_gather(yh.reshape(NSLOT, _HD), destc.reshape(-1), nchunks=4)
    out = _combine(gh.reshape(T, D), shared,
                   scale[:, 0:1], scale[:, 1:2])
    return (out.reshape(x_in.shape), aux.reshape(()), util.reshape(()))


# speedup vs baseline: 1.0578x; 1.0578x over previous
import dataclasses
import functools

import jax
import jax.numpy as jnp
from jax import lax
from jax.experimental import pallas as pl
from jax.experimental.pallas import tpu as pltpu
from jax.experimental.pallas import tpu_sc as plsc

T = 2048
D = 768
H = 1024
E = 8
CAP = 640
NSLOT = E * CAP
DISP_PAD = NSLOT + 16
NW = 32
_PREC = lax.Precision.DEFAULT


def _router_body(lg_ref, dest_ref, destc_ref, scale_ref, aux_ref,
                 util_ref):
    logits = lg_ref[...]
    m = jnp.max(logits, axis=1, keepdims=True)
    p = jnp.exp(logits - m)
    probs = p / jnp.sum(p, axis=1, keepdims=True)

    eio = lax.broadcasted_iota(jnp.int32, (T, E), 1)
    v1 = jnp.max(probs, axis=1, keepdims=True)
    a1 = jnp.min(jnp.where(probs == v1, eio, E), axis=1, keepdims=True)
    m1 = eio == a1
    pm = jnp.where(m1, -1.0, probs)
    v2 = jnp.max(pm, axis=1, keepdims=True)
    a2 = jnp.min(jnp.where(pm == v2, eio, E), axis=1, keepdims=True)
    m2 = eio == a2
    oh = jnp.logical_or(m1, m2)

    counts = jnp.sum(oh.astype(jnp.float32), axis=0, keepdims=True)
    util = jnp.sum((counts > 0.0).astype(jnp.float32)) * (100.0 / E)
    meanprob = jnp.sum(probs, axis=0, keepdims=True) * (1.0 / T)
    aux = E * jnp.sum(meanprob * (counts * (1.0 / T)))
    util_ref[...] = jnp.reshape(util, (1, 1))
    aux_ref[...] = jnp.reshape(aux, (1, 1))

    pos = oh.astype(jnp.int32)
    sh = 1
    while sh < T:
        shifted = jnp.concatenate(
            [jnp.zeros((sh, E), jnp.int32), pos[: T - sh]], axis=0)
        pos = pos + shifted
        sh *= 2
    kept = jnp.logical_and(oh, pos <= CAP)
    slot = eio * CAP + (pos - 1)

    wsum = v1 + v2
    s1 = v1 / wsum
    s2 = v2 / wsum

    def pick(mk):
        sel = jnp.logical_and(mk, kept)
        d = jnp.sum(jnp.where(sel, slot, 0), axis=1, keepdims=True)
        k = jnp.sum(sel.astype(jnp.int32), axis=1, keepdims=True) > 0
        return d, k

    d1, k1 = pick(m1)
    d2, k2 = pick(m2)
    dest_ref[...] = jnp.concatenate(
        [jnp.where(k1, d1, NSLOT), jnp.where(k2, d2, NSLOT)], axis=1)
    destc_ref[...] = jnp.concatenate(
        [jnp.where(k1, d1, 0), jnp.where(k2, d2, 0)], axis=1)
    scale_ref[...] = jnp.concatenate(
        [jnp.where(k1, s1, 0.0), jnp.where(k2, s2, 0.0)], axis=1)


def _router(logits):
    return pl.pallas_call(
        _router_body,
        out_shape=[
            jax.ShapeDtypeStruct((T, 2), jnp.int32),
            jax.ShapeDtypeStruct((T, 2), jnp.int32),
            jax.ShapeDtypeStruct((T, 2), jnp.float32),
            jax.ShapeDtypeStruct((1, 1), jnp.float32),
            jax.ShapeDtypeStruct((1, 1), jnp.float32),
        ],
    )(logits)


def _build_disp(dest_flat):
    mesh = plsc.VectorSubcoreMesh(core_axis_name="c", subcore_axis_name="s")
    cp = pltpu.CompilerParams()
    if "needs_layout_passes" in pltpu.CompilerParams.__dataclass_fields__:
        cp = dataclasses.replace(cp, needs_layout_passes=False)

    @functools.partial(
        pl.kernel,
        out_type=jax.ShapeDtypeStruct((NSLOT,), jnp.int32),
        mesh=mesh,
        compiler_params=cp,
        scratch_types=[
            pltpu.VMEM((2 * T,), jnp.int32),
            pltpu.VMEM((DISP_PAD,), jnp.int32),
        ],
    )
    def k(dest_hbm, disp_hbm, dest_v, disp_v):
        wid = lax.axis_index("s") * 2 + lax.axis_index("c")

        @pl.when(wid == 0)
        def _():
            pltpu.sync_copy(dest_hbm, dest_v)

            @pl.loop(0, DISP_PAD // 16)
            def _(i):
                disp_v[pl.ds(i * 16, 16)] = (
                    (i * 16 + lax.iota(jnp.int32, 16)) & (T - 1))

            @pl.loop(0, (2 * T) // 16)
            def _(i):
                idx16 = dest_v[pl.ds(i * 16, 16)]
                tok16 = lax.shift_right_logical(
                    i * 16 + lax.iota(jnp.int32, 16), 1)
                plsc.store_scatter(disp_v, [idx16], tok16)

            pltpu.sync_copy(disp_v.at[pl.ds(0, NSLOT)], disp_hbm)

    return k(dest_flat)


def _sc_gather(table, idx, nchunks):
    nrows, d = idx.shape[0], table.shape[1]
    per_w = nrows // NW
    chunk = per_w // nchunks
    assert chunk * nchunks == per_w and chunk <= 128 and chunk % 8 == 0
    mesh = plsc.VectorSubcoreMesh(core_axis_name="c", subcore_axis_name="s")

    @functools.partial(
        pl.kernel,
        out_type=jax.ShapeDtypeStruct((nrows, d), jnp.float32),
        mesh=mesh,
        scratch_types=(
            [pltpu.VMEM((nchunks, chunk), jnp.int32)]
            + [pltpu.VMEM((chunk, d), jnp.float32)] * nchunks
            + [pltpu.SemaphoreType.DMA] * (2 * nchunks)
        ),
    )
    def k(table_hbm, idx_hbm, out_hbm, idx_v, *bufs_and_sems):
        rows = bufs_and_sems[:nchunks]
        sg = bufs_and_sems[nchunks:2 * nchunks]
        so = bufs_and_sems[2 * nchunks:]
        wid = lax.axis_index("s") * 2 + lax.axis_index("c")
        base = wid * per_w
        for j in range(nchunks):
            pltpu.sync_copy(idx_hbm.at[pl.ds(base + j * chunk, chunk)],
                            idx_v.at[j])
        gs = [pltpu.async_copy(table_hbm.at[idx_v.at[j]], rows[j], sg[j])
              for j in range(nchunks)]
        os = []
        for j in range(nchunks):
            gs[j].wait()
            os.append(pltpu.async_copy(
                rows[j], out_hbm.at[pl.ds(base + j * chunk, chunk)], so[j]))
        for o in os:
            o.wait()

    return k(table, idx)


_HD = D // 2


def _pack_halves(a, b):
    ua = lax.bitcast_convert_type(a, jnp.uint32)
    ub = lax.bitcast_convert_type(b, jnp.uint32)
    pa = lax.shift_right_logical(ua + jnp.uint32(0x8000), jnp.uint32(16))
    pb = (ub + jnp.uint32(0x8000)) & jnp.uint32(0xFFFF0000)
    return lax.bitcast_convert_type(pa | pb, jnp.float32)


def _unpack_halves(p):
    up = lax.bitcast_convert_type(p, jnp.uint32)
    a = lax.bitcast_convert_type(lax.shift_left(up, jnp.uint32(16)),
                                 jnp.float32)
    b = lax.bitcast_convert_type(up & jnp.uint32(0xFFFF0000), jnp.float32)
    return jnp.concatenate([a, b], axis=1)


def _pack_x_body(x_ref, o_ref):
    xv = x_ref[...]
    o_ref[...] = _pack_halves(xv[:, :_HD], xv[:, _HD:])


def _pack_x(x):
    return pl.pallas_call(
        _pack_x_body,
        out_shape=jax.ShapeDtypeStruct((T, _HD), jnp.float32),
    )(x)


def _swiglu_block(xb, w1, w2, w3):
    h1 = lax.dot_general(xb, w1, (((1,), (1,)), ((), ())),
                         preferred_element_type=jnp.float32, precision=_PREC)
    h3 = lax.dot_general(xb, w3, (((1,), (1,)), ((), ())),
                         preferred_element_type=jnp.float32, precision=_PREC)
    g = h1 * (1.0 / (1.0 + jnp.exp(-h1))) * h3
    return lax.dot_general(g, w2, (((1,), (1,)), ((), ())),
                           preferred_element_type=jnp.float32,
                           precision=_PREC)


def _expert_ffn_body(x_ref, w1_ref, w2_ref, w3_ref, o_ref):
    xb = _unpack_halves(x_ref[0])
    y = _swiglu_block(xb, w1_ref[0], w2_ref[0], w3_ref[0])
    o_ref[0] = _pack_halves(y[:, :_HD], y[:, _HD:])


def _expert_ffn(xbufh3, w1s, w2s, w3s):
    return pl.pallas_call(
        _expert_ffn_body,
        grid=(E,),
        in_specs=[
            pl.BlockSpec((1, CAP, _HD), lambda e: (e, 0, 0)),
            pl.BlockSpec((1, H, D), lambda e: (e, 0, 0)),
            pl.BlockSpec((1, D, H), lambda e: (e, 0, 0)),
            pl.BlockSpec((1, H, D), lambda e: (e, 0, 0)),
        ],
        out_specs=pl.BlockSpec((1, CAP, _HD), lambda e: (e, 0, 0)),
        out_shape=jax.ShapeDtypeStruct((E, CAP, _HD), jnp.float32),
    )(xbufh3, w1s, w2s, w3s)


def _shared_ffn_body(x_ref, w1_ref, w2_ref, w3_ref, o_ref):
    o_ref[...] = _swiglu_block(x_ref[...], w1_ref[...], w2_ref[...],
                               w3_ref[...])


def _shared_ffn(x, sw1, sw2, sw3):
    nb = 8
    return pl.pallas_call(
        _shared_ffn_body,
        grid=(nb,),
        in_specs=[
            pl.BlockSpec((T // nb, D), lambda i: (i, 0)),
            pl.BlockSpec((H, D), lambda i: (0, 0)),
            pl.BlockSpec((D, H), lambda i: (0, 0)),
            pl.BlockSpec((H, D), lambda i: (0, 0)),
        ],
        out_specs=pl.BlockSpec((T // nb, D), lambda i: (i, 0)),
        out_shape=jax.ShapeDtypeStruct((T, D), jnp.float32),
    )(x, sw1, sw2, sw3)


def _combine_body(g_ref, sh_ref, s0_ref, s1_ref, o_ref):
    g = g_ref[...]
    g0 = _unpack_halves(g[:, :_HD])
    g1 = _unpack_halves(g[:, _HD:])
    o_ref[...] = sh_ref[...] + s0_ref[...] * g0 + s1_ref[...] * g1


def _combine(g2, shared, s0, s1):
    nb = 4
    return pl.pallas_call(
        _combine_body,
        grid=(nb,),
        in_specs=[
            pl.BlockSpec((T // nb, D), lambda i: (i, 0)),
            pl.BlockSpec((T // nb, D), lambda i: (i, 0)),
            pl.BlockSpec((T // nb, 1), lambda i: (i, 0)),
            pl.BlockSpec((T // nb, 1), lambda i: (i, 0)),
        ],
        out_specs=pl.BlockSpec((T // nb, D), lambda i: (i, 0)),
        out_shape=jax.ShapeDtypeStruct((T, D), jnp.float32),
    )(g2, shared, s0, s1)


def kernel(x_in, router_w, w1s, w2s, w3s, sw1, sw2, sw3):
    x = x_in.reshape(T, D)
    logits = x @ router_w.T
    dest, destc, scale, aux, util = _router(logits)
    disp = _build_disp(dest.reshape(-1))
    xh = _pack_x(x)
    xbufh = _sc_gather(xh, disp, nchunks=4)
    shared = _shared_ffn(x, sw1, sw2, sw3)
    yh = _expert_ffn(xbufh.reshape(E, CAP, _HD), w1s, w2s, w3s)
    gh = _sc_gather(yh.reshape(NSLOT, _HD), destc.reshape(-1), nchunks=4)
    out = _combine(gh.reshape(T, D), shared,
                   scale[:, 0:1], scale[:, 1:2])
    return (out.reshape(x_in.shape), aux.reshape(()), util.reshape(()))


# speedup vs baseline: 1.1737x; 1.1096x over previous
import dataclasses
import functools

import jax
import jax.numpy as jnp
from jax import lax
from jax.experimental import pallas as pl
from jax.experimental.pallas import tpu as pltpu
from jax.experimental.pallas import tpu_sc as plsc

T = 2048
D = 768
H = 1024
E = 8
CAP = 640
NSLOT = E * CAP
DISP_PAD = NSLOT + 16
NW = 32
_PREC = lax.Precision.DEFAULT


def _router_body(lg_ref, dest_ref, scale_ref, aux_ref, util_ref):
    logits = lg_ref[...]
    m = jnp.max(logits, axis=1, keepdims=True)
    p = jnp.exp(logits - m)
    probs = p / jnp.sum(p, axis=1, keepdims=True)

    eio = lax.broadcasted_iota(jnp.int32, (T, E), 1)
    v1 = jnp.max(probs, axis=1, keepdims=True)
    a1 = jnp.min(jnp.where(probs == v1, eio, E), axis=1, keepdims=True)
    m1 = eio == a1
    pm = jnp.where(m1, -1.0, probs)
    v2 = jnp.max(pm, axis=1, keepdims=True)
    a2 = jnp.min(jnp.where(pm == v2, eio, E), axis=1, keepdims=True)
    m2 = eio == a2
    oh = jnp.logical_or(m1, m2)

    counts = jnp.sum(oh.astype(jnp.float32), axis=0, keepdims=True)
    util = jnp.sum((counts > 0.0).astype(jnp.float32)) * (100.0 / E)
    meanprob = jnp.sum(probs, axis=0, keepdims=True) * (1.0 / T)
    aux = E * jnp.sum(meanprob * (counts * (1.0 / T)))
    util_ref[...] = jnp.reshape(util, (1, 1))
    aux_ref[...] = jnp.reshape(aux, (1, 1))

    pos = oh.astype(jnp.int32)
    sh = 1
    while sh < T:
        shifted = jnp.concatenate(
            [jnp.zeros((sh, E), jnp.int32), pos[: T - sh]], axis=0)
        pos = pos + shifted
        sh *= 2
    kept = jnp.logical_and(oh, pos <= CAP)
    slot = eio * CAP + (pos - 1)

    wsum = v1 + v2
    s1 = v1 / wsum
    s2 = v2 / wsum

    def pick(mk):
        sel = jnp.logical_and(mk, kept)
        d = jnp.sum(jnp.where(sel, slot, 0), axis=1, keepdims=True)
        k = jnp.sum(sel.astype(jnp.int32), axis=1, keepdims=True) > 0
        return d, k

    d1, k1 = pick(m1)
    d2, k2 = pick(m2)
    dest_ref[...] = jnp.concatenate(
        [jnp.where(k1, d1, NSLOT), jnp.where(k2, d2, NSLOT)], axis=1)
    scale_ref[...] = jnp.concatenate(
        [jnp.where(k1, s1, 0.0), jnp.where(k2, s2, 0.0)], axis=1)


def _router(logits):
    return pl.pallas_call(
        _router_body,
        out_shape=[
            jax.ShapeDtypeStruct((T, 2), jnp.int32),
            jax.ShapeDtypeStruct((T, 2), jnp.float32),
            jax.ShapeDtypeStruct((1, 1), jnp.float32),
            jax.ShapeDtypeStruct((1, 1), jnp.float32),
        ],
    )(logits)


def _build_disp(dest_flat):
    mesh = plsc.VectorSubcoreMesh(core_axis_name="c", subcore_axis_name="s")
    cp = pltpu.CompilerParams()
    if "needs_layout_passes" in pltpu.CompilerParams.__dataclass_fields__:
        cp = dataclasses.replace(cp, needs_layout_passes=False)

    @functools.partial(
        pl.kernel,
        out_type=[jax.ShapeDtypeStruct((NSLOT,), jnp.int32),
                  jax.ShapeDtypeStruct((2 * T,), jnp.int32)],
        mesh=mesh,
        compiler_params=cp,
        scratch_types=[
            pltpu.VMEM((2 * T,), jnp.int32),
            pltpu.VMEM((DISP_PAD,), jnp.int32),
            pltpu.VMEM((2 * T,), jnp.int32),
        ],
    )
    def k(dest_hbm, disp_hbm, destc_hbm, dest_v, disp_v, destc_v):
        wid = lax.axis_index("s") * 2 + lax.axis_index("c")

        @pl.when(wid == 0)
        def _():
            pltpu.sync_copy(dest_hbm, dest_v)

            @pl.loop(0, DISP_PAD // 16)
            def _(i):
                disp_v[pl.ds(i * 16, 16)] = (
                    (i * 16 + lax.iota(jnp.int32, 16)) & (T - 1))

            @pl.loop(0, (2 * T) // 16)
            def _(i):
                j16 = i * 16 + lax.iota(jnp.int32, 16)
                tok16 = lax.shift_right_logical(j16, 1)
                slot16 = dest_v[pl.ds(i * 16, 16)]
                plsc.store_scatter(disp_v, [slot16], tok16)

            pltpu.sync_copy(disp_v.at[pl.ds(0, NSLOT)], disp_hbm)

        @pl.when(wid == 1)
        def _():
            pltpu.sync_copy(dest_hbm, dest_v)

            @pl.loop(0, (2 * T) // 16)
            def _(i):
                j16 = i * 16 + lax.iota(jnp.int32, 16)
                tok16 = j16 & (T - 1)
                k16 = lax.shift_right_logical(j16, 11)
                src16 = lax.shift_left(tok16, 1) + k16
                v16 = plsc.load_gather(dest_v, [src16])
                destc_v[pl.ds(i * 16, 16)] = jnp.where(v16 >= NSLOT, 0, v16)

            pltpu.sync_copy(destc_v, destc_hbm)

    return k(dest_flat)


def _sc_gather(table, idx, nchunks):
    nrows, d = idx.shape[0], table.shape[1]
    per_w = nrows // NW
    chunk = per_w // nchunks
    assert chunk * nchunks == per_w and chunk <= 128 and chunk % 8 == 0
    mesh = plsc.VectorSubcoreMesh(core_axis_name="c", subcore_axis_name="s")

    @functools.partial(
        pl.kernel,
        out_type=jax.ShapeDtypeStruct((nrows, d), jnp.float32),
        mesh=mesh,
        scratch_types=(
            [pltpu.VMEM((nchunks, chunk), jnp.int32)]
            + [pltpu.VMEM((chunk, d), jnp.float32)] * nchunks
            + [pltpu.SemaphoreType.DMA] * (2 * nchunks)
        ),
    )
    def k(table_hbm, idx_hbm, out_hbm, idx_v, *bufs_and_sems):
        rows = bufs_and_sems[:nchunks]
        sg = bufs_and_sems[nchunks:2 * nchunks]
        so = bufs_and_sems[2 * nchunks:]
        wid = lax.axis_index("s") * 2 + lax.axis_index("c")
        base = wid * per_w
        for j in range(nchunks):
            pltpu.sync_copy(idx_hbm.at[pl.ds(base + j * chunk, chunk)],
                            idx_v.at[j])
        gs = [pltpu.async_copy(table_hbm.at[idx_v.at[j]], rows[j], sg[j])
              for j in range(nchunks)]
        os = []
        for j in range(nchunks):
            gs[j].wait()
            os.append(pltpu.async_copy(
                rows[j], out_hbm.at[pl.ds(base + j * chunk, chunk)], so[j]))
        for o in os:
            o.wait()

    return k(table, idx)


_HD = D // 2


def _pack_halves(a, b):
    ua = lax.bitcast_convert_type(a, jnp.uint32)
    ub = lax.bitcast_convert_type(b, jnp.uint32)
    pa = lax.shift_right_logical(ua + jnp.uint32(0x8000), jnp.uint32(16))
    pb = (ub + jnp.uint32(0x8000)) & jnp.uint32(0xFFFF0000)
    return lax.bitcast_convert_type(pa | pb, jnp.float32)


def _unpack_halves(p):
    up = lax.bitcast_convert_type(p, jnp.uint32)
    a = lax.bitcast_convert_type(lax.shift_left(up, jnp.uint32(16)),
                                 jnp.float32)
    b = lax.bitcast_convert_type(up & jnp.uint32(0xFFFF0000), jnp.float32)
    return jnp.concatenate([a, b], axis=1)


def _pack_x_body(x_ref, o_ref):
    xv = x_ref[...]
    o_ref[...] = _pack_halves(xv[:, :_HD], xv[:, _HD:])


def _pack_x(x):
    return pl.pallas_call(
        _pack_x_body,
        out_shape=jax.ShapeDtypeStruct((T, _HD), jnp.float32),
    )(x)


def _swiglu_block(xb, w1, w2, w3):
    h1 = lax.dot_general(xb, w1, (((1,), (1,)), ((), ())),
                         preferred_element_type=jnp.float32, precision=_PREC)
    h3 = lax.dot_general(xb, w3, (((1,), (1,)), ((), ())),
                         preferred_element_type=jnp.float32, precision=_PREC)
    g = h1 * (1.0 / (1.0 + jnp.exp(-h1))) * h3
    return lax.dot_general(g, w2, (((1,), (1,)), ((), ())),
                           preferred_element_type=jnp.float32,
                           precision=_PREC)


def _expert_ffn_body(x_ref, w1_ref, w2_ref, w3_ref, o_ref):
    xb = _unpack_halves(x_ref[0])
    y = _swiglu_block(xb, w1_ref[0], w2_ref[0], w3_ref[0])
    o_ref[0] = _pack_halves(y[:, :_HD], y[:, _HD:])


def _expert_ffn(xbufh3, w1s, w2s, w3s):
    return pl.pallas_call(
        _expert_ffn_body,
        grid=(E,),
        in_specs=[
            pl.BlockSpec((1, CAP, _HD), lambda e: (e, 0, 0)),
            pl.BlockSpec((1, H, D), lambda e: (e, 0, 0)),
            pl.BlockSpec((1, D, H), lambda e: (e, 0, 0)),
            pl.BlockSpec((1, H, D), lambda e: (e, 0, 0)),
        ],
        out_specs=pl.BlockSpec((1, CAP, _HD), lambda e: (e, 0, 0)),
        out_shape=jax.ShapeDtypeStruct((E, CAP, _HD), jnp.float32),
    )(xbufh3, w1s, w2s, w3s)


def _shared_ffn_body(x_ref, w1_ref, w2_ref, w3_ref, o_ref):
    o_ref[...] = _swiglu_block(x_ref[...], w1_ref[...], w2_ref[...],
                               w3_ref[...])


def _shared_ffn(x, sw1, sw2, sw3):
    nb = 4
    return pl.pallas_call(
        _shared_ffn_body,
        grid=(nb,),
        in_specs=[
            pl.BlockSpec((T // nb, D), lambda i: (i, 0)),
            pl.BlockSpec((H, D), lambda i: (0, 0)),
            pl.BlockSpec((D, H), lambda i: (0, 0)),
            pl.BlockSpec((H, D), lambda i: (0, 0)),
        ],
        out_specs=pl.BlockSpec((T // nb, D), lambda i: (i, 0)),
        out_shape=jax.ShapeDtypeStruct((T, D), jnp.float32),
    )(x, sw1, sw2, sw3)


def _combine_body(g0_ref, g1_ref, sh_ref, s_ref, o_ref):
    g0 = _unpack_halves(g0_ref[...])
    g1 = _unpack_halves(g1_ref[...])
    s = s_ref[...]
    o_ref[...] = sh_ref[...] + s[:, 0:1] * g0 + s[:, 1:2] * g1


def _combine(gh, shared, scale):
    nb = 4
    return pl.pallas_call(
        _combine_body,
        grid=(nb,),
        in_specs=[
            pl.BlockSpec((T // nb, _HD), lambda i: (i, 0)),
            pl.BlockSpec((T // nb, _HD), lambda i: (i + nb, 0)),
            pl.BlockSpec((T // nb, D), lambda i: (i, 0)),
            pl.BlockSpec((T // nb, 2), lambda i: (i, 0)),
        ],
        out_specs=pl.BlockSpec((T // nb, D), lambda i: (i, 0)),
        out_shape=jax.ShapeDtypeStruct((T, D), jnp.float32),
    )(gh, gh, shared, scale)


def kernel(x_in, router_w, w1s, w2s, w3s, sw1, sw2, sw3):
    x = x_in.reshape(T, D)
    logits = x @ router_w.T
    dest, scale, aux, util = _router(logits)
    disp, destc = _build_disp(dest.reshape(-1))
    xh = _pack_x(x)
    xbufh = _sc_gather(xh, disp, nchunks=4)
    shared = _shared_ffn(x, sw1, sw2, sw3)
    yh = _expert_ffn(xbufh.reshape(E, CAP, _HD), w1s, w2s, w3s)
    gh = _sc_gather(yh.reshape(NSLOT, _HD), destc, nchunks=4)
    out = _combine(gh, shared, scale)
    return (out.reshape(x_in.shape), aux.reshape(()), util.reshape(()))
